# baseline (speedup 1.0000x reference)
import functools

import jax
import jax.numpy as jnp
import numpy as np
from jax import lax
from jax.experimental import pallas as pl
from jax.experimental.pallas import tpu as pltpu
from jax.experimental.pallas import tpu_sc as plsc

B = 16384
EMB_K = 16
ROWS_PER_128 = 8
TBL_ROWS = 1000000 // ROWS_PER_128
NC = 2
NS = 16
NW = NC * NS
BPW = B // NW
CHUNK = 128
NCHUNK = BPW // CHUNK
LANE = 16


def _gather_body(uidx_hbm, iidx_hbm, w_hbm, h_hbm, uout_hbm, vout_hbm,
                 idx_v, hi_v, rows_v, sem):
    wid = lax.axis_index("s") * NC + lax.axis_index("c")
    base = wid * BPW
    for t in range(2):
        src_idx = uidx_hbm if t == 0 else iidx_hbm
        tbl = w_hbm if t == 0 else h_hbm
        out = uout_hbm if t == 0 else vout_hbm
        pltpu.sync_copy(src_idx.at[pl.ds(base, BPW)], idx_v)
        for i in range(BPW // LANE):
            sl = pl.ds(i * LANE, LANE)
            hi_v[sl] = lax.shift_right_logical(idx_v[sl], 3)
        copies = [
            pltpu.async_copy(
                tbl.at[hi_v.at[pl.ds(j * CHUNK, CHUNK)]],
                rows_v.at[pl.ds(j * CHUNK, CHUNK)],
                sem,
            )
            for j in range(NCHUNK)
        ]
        for c in copies:
            c.wait()
        pltpu.sync_copy(rows_v, out.at[pl.ds(base, BPW)])


_gather = functools.partial(
    pl.kernel,
    mesh=plsc.VectorSubcoreMesh(core_axis_name="c", subcore_axis_name="s"),
    out_type=[
        jax.ShapeDtypeStruct((B, 128), jnp.float32),
        jax.ShapeDtypeStruct((B, 128), jnp.float32),
    ],
    scratch_types=[
        pltpu.VMEM((BPW,), jnp.int32),
        pltpu.VMEM((BPW,), jnp.int32),
        pltpu.VMEM((BPW, 128), jnp.float32),
        pltpu.SemaphoreType.DMA,
    ],
)(_gather_body)


BLK = 2048


def _mlp_body(x_ref, u_ref, v_ref, sel_ref, w1u_ref, w1v_ref, b1_ref,
              w2t_ref, o_ref):
    lane_blk = lax.broadcasted_iota(jnp.int32, (BLK, 128), 1) >> 4
    su = x_ref[...][:, 0:1] & 7
    si = x_ref[...][:, 1:2] & 7
    mu = (lane_blk == su).astype(jnp.float32)
    mi = (lane_blk == si).astype(jnp.float32)
    sel = sel_ref[...]
    zu = jnp.dot(u_ref[...] * mu, sel, preferred_element_type=jnp.float32)
    zv = jnp.dot(v_ref[...] * mi, sel, preferred_element_type=jnp.float32)
    h = (
        jnp.dot(zu, w1u_ref[...], preferred_element_type=jnp.float32)
        + jnp.dot(zv, w1v_ref[...], preferred_element_type=jnp.float32)
        + b1_ref[...]
    )
    h = jnp.maximum(h, 0.0)
    o_ref[...] = jnp.dot(h, w2t_ref[...], preferred_element_type=jnp.float32)


def _mlp(x, u128, v128, sel, w1u, w1v, b1_2d, w2t):
    grid = B // BLK
    return pl.pallas_call(
        _mlp_body,
        grid=(grid,),
        in_specs=[
            pl.BlockSpec((BLK, 2), lambda i: (i, 0)),
            pl.BlockSpec((BLK, 128), lambda i: (i, 0)),
            pl.BlockSpec((BLK, 128), lambda i: (i, 0)),
            pl.BlockSpec((128, EMB_K), lambda i: (0, 0)),
            pl.BlockSpec((EMB_K, EMB_K), lambda i: (0, 0)),
            pl.BlockSpec((EMB_K, EMB_K), lambda i: (0, 0)),
            pl.BlockSpec((1, EMB_K), lambda i: (0, 0)),
            pl.BlockSpec((EMB_K, 1), lambda i: (0, 0)),
        ],
        out_specs=pl.BlockSpec((BLK, 1), lambda i: (i, 0)),
        out_shape=jax.ShapeDtypeStruct((B, 1), jnp.float32),
    )(x, u128, v128, sel, w1u, w1v, b1_2d, w2t)


_SEL = jnp.asarray(
    np.equal(np.arange(128)[:, None] % EMB_K, np.arange(EMB_K)[None, :]),
    dtype=jnp.float32,
)


@jax.jit
def kernel(x, W, H, W1, b1, W2):
    user_idx = x[:, 0]
    item_idx = x[:, 1]
    W128 = W.reshape(TBL_ROWS, 128)
    H128 = H.reshape(TBL_ROWS, 128)
    U128, V128 = _gather(user_idx, item_idx, W128, H128)
    w1u = W1[:, :EMB_K].T
    w1v = W1[:, EMB_K:].T
    return _mlp(x, U128, V128, _SEL, w1u, w1v, b1.reshape(1, EMB_K), W2.T)
